# baseline (speedup 1.0000x reference)
import functools

import jax
import jax.numpy as jnp
from jax import lax
from jax.experimental import pallas as pl
from jax.experimental.pallas import tpu as pltpu
from jax.experimental.pallas import tpu_sc as plsc

_N = 10000
_NPAD = 10240
_D = 128
_NC = 2
_NS = 16
_NW = _NC * _NS
_CHUNK = 80
_NBUF = 5
_BR = 256
_EPS = 1e-5


def _deg_body(dst2_hbm, zerosn_hbm, out_hbm, didx, hist, *, epw):
    cid = lax.axis_index("c")
    sid = lax.axis_index("s")
    wid = cid * _NS + sid
    nch = epw // _CHUNK
    pltpu.sync_copy(dst2_hbm.at[wid], didx)
    pltpu.sync_copy(zerosn_hbm, hist)
    ones16 = jnp.ones((16,), jnp.float32)

    def body(j, carry):
        for k in range(_CHUNK // 16):
            idx16 = didx[j, pl.ds(k * 16, 16)]
            plsc.addupdate_scatter(hist, [idx16], ones16)
        return carry

    lax.fori_loop(0, nch, body, 0)
    pltpu.sync_copy(hist, out_hbm.at[wid])


def _agg_body(s2_hbm, src3_hbm, dst3_hbm, out_hbm,
              sidx, didx, r0b, r1b, r2b, r3b, r4b, acc, sem_g, sem_s, *, epw):
    rows = [r0b, r1b, r2b, r3b, r4b]
    nbuf = len(rows)
    lag = 2
    cid = lax.axis_index("c")
    sid = lax.axis_index("s")
    rpt = _NPAD // _NS
    r0 = sid * rpt
    nch = epw // _CHUNK
    half = s2_hbm.at[cid]

    pltpu.sync_copy(half.at[pl.ds(r0, rpt)], acc.at[pl.ds(r0, rpt)])

    pltpu.sync_copy(src3_hbm.at[sid], sidx)
    pltpu.sync_copy(dst3_hbm.at[sid], didx)
    plsc.subcore_barrier()

    for b in range(nbuf - lag):
        pltpu.async_copy(half.at[sidx.at[b]], rows[b], sem_g)

    ngrp = nch // nbuf

    def grp_body(g, carry):
        for b in range(nbuf):
            j = g * nbuf + b
            pltpu.make_async_copy(half.at[sidx.at[b]], rows[b],
                                  sem_g).wait()
            pltpu.async_copy(rows[b], acc.at[didx.at[j]], sem_s, add=True)

            @pl.when(j >= lag)
            def _():
                pltpu.make_async_copy(rows[b], acc.at[didx.at[j]],
                                      sem_s).wait()

            @pl.when(j + nbuf - lag < nch)
            def _():
                nb = (b + nbuf - lag) % nbuf
                pltpu.async_copy(half.at[sidx.at[j + nbuf - lag]], rows[nb],
                                 sem_g)
        return carry

    lax.fori_loop(0, ngrp, grp_body, 0)
    for _ in range(lag):
        pltpu.make_async_copy(rows[0], acc.at[didx.at[0]], sem_s).wait()
    plsc.subcore_barrier()
    pltpu.sync_copy(acc.at[pl.ds(r0, rpt)], out_hbm.at[cid, pl.ds(r0, rpt)])


def _init_body(x_ref, w_ref, b_ref, degt_ref, wc_ref, h_ref, s_ref,
               dinv_ref):
    h = jnp.dot(x_ref[...], w_ref[...], preferred_element_type=jnp.float32)
    h = jnp.maximum(h + b_ref[...], 0.0)
    h_ref[...] = h
    deg = jnp.sum(degt_ref[...], axis=1, keepdims=True) + 1.0
    dinv = lax.rsqrt(deg)
    dinv_ref[...] = dinv
    hw = jnp.dot(h, wc_ref[...], preferred_element_type=jnp.float32) * dinv
    s_ref[0] = hw[:, :_D // 2]
    s_ref[1] = hw[:, _D // 2:]


def _bn_scale_shift(st, gb):
    mu = st[0:1] / float(_N)
    var = st[1:2] / float(_N) - mu * mu
    a = gb[0:1] * lax.rsqrt(var + _EPS)
    return a, gb[1:2] - mu * a


def _phase0(p_ref, dinv_ref, b_ref, y_scr, st_scr):
    i = pl.program_id(1)
    pb = p_ref[...]
    y = jnp.concatenate([pb[0], pb[1]], axis=1) * dinv_ref[...] + b_ref[...]
    y_scr[pl.ds(i * _BR, _BR), :] = y
    rowid = lax.broadcasted_iota(jnp.int32, (_BR, 1), 0) + i * _BR
    ym = jnp.where(rowid < _N, y, 0.0)
    snew = jnp.concatenate([jnp.sum(ym, axis=0, keepdims=True),
                            jnp.sum(ym * ym, axis=0, keepdims=True)], axis=0)

    @pl.when(i == 0)
    def _():
        st_scr[...] = snew

    @pl.when(i > 0)
    def _():
        st_scr[...] = st_scr[...] + snew


def _conv_epi_body(*refs, has_res):
    if has_res:
        (p_ref, dinv_ref, b_ref, gb_ref, wc_ref, hin_ref,
         h_ref, s_ref, y_scr, st_scr) = refs
    else:
        (p_ref, dinv_ref, b_ref, gb_ref, wc_ref,
         h_ref, s_ref, y_scr, st_scr) = refs
    ph = pl.program_id(0)
    i = pl.program_id(1)

    @pl.when(ph == 0)
    def _():
        _phase0(p_ref, dinv_ref, b_ref, y_scr, st_scr)

    @pl.when(ph == 1)
    def _():
        y = y_scr[pl.ds(i * _BR, _BR), :]
        a, c = _bn_scale_shift(st_scr[...], gb_ref[...])
        h = jnp.maximum(y * a + c, 0.0)
        if has_res:
            h = h + hin_ref[...]
        h_ref[...] = h
        hw = jnp.dot(h, wc_ref[...], preferred_element_type=jnp.float32)
        hw = hw * dinv_ref[...]
        s_ref[0] = hw[:, :_D // 2]
        s_ref[1] = hw[:, _D // 2:]


def _conv_head_body(p_ref, dinv_ref, b_ref, gb_ref, hin_ref, w1_ref, b1_ref,
                    w2_ref, b2_ref, o_ref, y_scr, st_scr):
    ph = pl.program_id(0)
    i = pl.program_id(1)

    @pl.when(ph == 0)
    def _():
        _phase0(p_ref, dinv_ref, b_ref, y_scr, st_scr)

    @pl.when(ph == 1)
    def _():
        y = y_scr[pl.ds(i * _BR, _BR), :]
        a, c = _bn_scale_shift(st_scr[...], gb_ref[...])
        h = jnp.maximum(y * a + c, 0.0) + hin_ref[...]
        r = jnp.dot(h, w1_ref[...], preferred_element_type=jnp.float32)
        r = jnp.maximum(r + b1_ref[...], 0.0)
        o = jnp.sum(r * w2_ref[...], axis=1, keepdims=True) + b2_ref[...]
        o_ref[...] = o


def _row_spec(i_map=lambda i: (i, 0), shape=(_BR, _D)):
    return pl.BlockSpec(shape, i_map)


_FULL = lambda i: (0, 0)


def kernel(x, edge_index, W_in, b_in, conv_W, conv_b, bn_g, bn_b,
           Wo1, bo1, Wo2, bo2):
    n, d = x.shape
    e = edge_index.shape[1]
    epw = e // _NW
    assert n == _N and d == _D and e % (_NW * _CHUNK) == 0

    ept = e // _NS
    assert ept % (_CHUNK * 5) == 0
    src = edge_index[0]
    dst = edge_index[1]
    src2 = src.reshape(_NW, epw // _CHUNK, _CHUNK)
    dst2 = dst.reshape(_NW, epw // _CHUNK, _CHUNK)
    src3 = src.reshape(_NS, ept // _CHUNK, _CHUNK)
    dst3 = dst.reshape(_NS, ept // _CHUNK, _CHUNK)
    xp = jnp.pad(x, ((0, _NPAD - n), (0, 0)))
    zerosn = jnp.zeros((_NPAD,), jnp.float32)

    mesh = plsc.VectorSubcoreMesh(core_axis_name="c", subcore_axis_name="s",
                                  num_cores=_NC, num_subcores=_NS)

    deg_call = pl.kernel(
        functools.partial(_deg_body, epw=epw),
        out_type=jax.ShapeDtypeStruct((_NW, _NPAD), jnp.float32),
        mesh=mesh,
        compiler_params=pltpu.CompilerParams(needs_layout_passes=False),
        scratch_types=[
            pltpu.VMEM((epw // _CHUNK, _CHUNK), jnp.int32),
            pltpu.VMEM((_NPAD,), jnp.float32),
        ],
    )
    agg_call = pl.kernel(
        functools.partial(_agg_body, epw=ept),
        out_type=jax.ShapeDtypeStruct((_NC, _NPAD, d // 2), jnp.float32),
        mesh=mesh,
        compiler_params=pltpu.CompilerParams(use_tc_tiling_on_sc=False),
        scratch_types=[
            pltpu.VMEM((ept // _CHUNK, _CHUNK), jnp.int32),
            pltpu.VMEM((ept // _CHUNK, _CHUNK), jnp.int32),
        ] + [pltpu.VMEM((_CHUNK, d // 2), jnp.float32)] * 5 + [
            pltpu.VMEM_SHARED((_NPAD, d // 2), jnp.float32),
            pltpu.SemaphoreType.DMA,
            pltpu.SemaphoreType.DMA,
        ],
    )

    nblk = _NPAD // _BR
    degp = deg_call(dst2, zerosn)
    degt = degp.T

    init_call = pl.pallas_call(
        _init_body,
        grid=(nblk,),
        in_specs=[_row_spec(),
                  pl.BlockSpec((_D, _D), _FULL),
                  pl.BlockSpec((1, _D), _FULL),
                  _row_spec(shape=(_BR, _NW)),
                  pl.BlockSpec((_D, _D), _FULL)],
        out_specs=[_row_spec(),
                   pl.BlockSpec((2, _BR, _D // 2), lambda i: (0, i, 0)),
                   pl.BlockSpec((_BR, 1), lambda i: (i, 0))],
        out_shape=[jax.ShapeDtypeStruct((_NPAD, _D), jnp.float32),
                   jax.ShapeDtypeStruct((2, _NPAD, _D // 2), jnp.float32),
                   jax.ShapeDtypeStruct((_NPAD, 1), jnp.float32)],
    )
    h, s, dinv = init_call(xp, W_in, b_in.reshape(1, _D), degt, conv_W[0])

    dmid = Wo1.shape[1]
    out = None
    for i in range(3):
        p = agg_call(s, src3, dst3)
        gb = jnp.stack([bn_g[i], bn_b[i]])
        base_specs = [
            pl.BlockSpec((_NC, _BR, _D // 2), lambda ph, ib: (0, ib, 0)),
            pl.BlockSpec((_BR, 1), lambda ph, ib: (ib, 0)),
            pl.BlockSpec((1, _D), lambda ph, ib: (0, 0)),
            pl.BlockSpec((2, _D), lambda ph, ib: (0, 0)),
        ]
        scr = [pltpu.VMEM((_NPAD, _D), jnp.float32),
               pltpu.VMEM((2, _D), jnp.float32)]
        if i < 2:
            has_res = i > 0
            specs = base_specs + [
                pl.BlockSpec((_D, _D), lambda ph, ib: (0, 0))]
            args = [p, dinv, conv_b[i].reshape(1, _D), gb, conv_W[i + 1]]
            if has_res:
                specs.append(pl.BlockSpec((_BR, _D), lambda ph, ib: (ib, 0)))
                args.append(h)
            fused_call = pl.pallas_call(
                functools.partial(_conv_epi_body, has_res=has_res),
                grid=(2, nblk),
                in_specs=specs,
                out_specs=[pl.BlockSpec((_BR, _D), lambda ph, ib: (ib, 0)),
                           pl.BlockSpec((2, _BR, _D // 2),
                                        lambda ph, ib: (0, ib, 0))],
                out_shape=[jax.ShapeDtypeStruct((_NPAD, _D), jnp.float32),
                           jax.ShapeDtypeStruct((2, _NPAD, _D // 2),
                                                jnp.float32)],
                scratch_shapes=scr,
            )
            h, s = fused_call(*args)
        else:
            specs = base_specs + [
                pl.BlockSpec((_BR, _D), lambda ph, ib: (ib, 0)),
                pl.BlockSpec((_D, dmid), lambda ph, ib: (0, 0)),
                pl.BlockSpec((1, dmid), lambda ph, ib: (0, 0)),
                pl.BlockSpec((1, dmid), lambda ph, ib: (0, 0)),
                pl.BlockSpec((1, 1), lambda ph, ib: (0, 0))]
            fused_call = pl.pallas_call(
                _conv_head_body,
                grid=(2, nblk),
                in_specs=specs,
                out_specs=[pl.BlockSpec((_BR, 1), lambda ph, ib: (ib, 0))],
                out_shape=[jax.ShapeDtypeStruct((_NPAD, 1), jnp.float32)],
                scratch_shapes=scr,
            )
            (o2,) = fused_call(p, dinv, conv_b[i].reshape(1, _D), gb, h,
                               Wo1, bo1.reshape(1, dmid),
                               Wo2.reshape(1, dmid), bo2.reshape(1, 1))
            out = o2[:n, 0]
    return out


# speedup vs baseline: 1.0262x; 1.0262x over previous
import functools

import jax
import jax.numpy as jnp
from jax import lax
from jax.experimental import pallas as pl
from jax.experimental.pallas import tpu as pltpu
from jax.experimental.pallas import tpu_sc as plsc

_N = 10000
_NPAD = 10240
_D = 128
_NC = 2
_NS = 16
_NW = _NC * _NS
_CHUNK = 80
_NBUF = 5
_BR = 256
_EPS = 1e-5


def _deg_body(dst2_hbm, zerosn_hbm, out_hbm, didx, hist, *, epw):
    cid = lax.axis_index("c")
    sid = lax.axis_index("s")
    wid = cid * _NS + sid
    nch = epw // _CHUNK
    pltpu.sync_copy(dst2_hbm.at[wid], didx)
    pltpu.sync_copy(zerosn_hbm, hist)
    ones16 = jnp.ones((16,), jnp.float32)

    def body(j, carry):
        for k in range(_CHUNK // 16):
            idx16 = didx[j, pl.ds(k * 16, 16)]
            plsc.addupdate_scatter(hist, [idx16], ones16)
        return carry

    lax.fori_loop(0, nch, body, 0)
    pltpu.sync_copy(hist, out_hbm.at[wid])


def _agg_body(s2_hbm, src3_hbm, dst3_hbm, out_hbm,
              sidx, didx, r0b, r1b, r2b, r3b, r4b, acc, sem_g, sem_s, *, epw):
    rows = [r0b, r1b, r2b, r3b, r4b]
    nbuf = len(rows)
    lag = 2
    cid = lax.axis_index("c")
    sid = lax.axis_index("s")
    rpt = _NPAD // _NS
    r0 = sid * rpt
    nch = epw // _CHUNK
    half = s2_hbm.at[cid]

    pltpu.sync_copy(half.at[pl.ds(r0, rpt)], acc.at[pl.ds(r0, rpt)])

    pltpu.sync_copy(src3_hbm.at[sid], sidx)
    pltpu.sync_copy(dst3_hbm.at[sid], didx)
    plsc.subcore_barrier()

    for b in range(nbuf - lag):
        pltpu.async_copy(half.at[sidx.at[b]], rows[b], sem_g)

    ngrp = nch // nbuf

    def grp_body(g, carry):
        for b in range(nbuf):
            j = g * nbuf + b
            pltpu.make_async_copy(half.at[sidx.at[b]], rows[b],
                                  sem_g).wait()
            pltpu.async_copy(rows[b], acc.at[didx.at[j]], sem_s, add=True)

            @pl.when(j >= lag)
            def _():
                pltpu.make_async_copy(rows[b], acc.at[didx.at[j]],
                                      sem_s).wait()

            @pl.when(j + nbuf - lag < nch)
            def _():
                nb = (b + nbuf - lag) % nbuf
                pltpu.async_copy(half.at[sidx.at[j + nbuf - lag]], rows[nb],
                                 sem_g)
        return carry

    lax.fori_loop(0, ngrp, grp_body, 0)
    for _ in range(lag):
        pltpu.make_async_copy(rows[0], acc.at[didx.at[0]], sem_s).wait()
    plsc.subcore_barrier()
    pltpu.sync_copy(acc.at[pl.ds(r0, rpt)], out_hbm.at[cid, pl.ds(r0, rpt)])


def _init_body(x_ref, w_ref, b_ref, degt_ref, wc_ref, h_ref, s_ref,
               dinv_ref):
    h = jnp.dot(x_ref[...], w_ref[...], preferred_element_type=jnp.float32)
    h = jnp.maximum(h + b_ref[...], 0.0)
    h_ref[...] = h
    deg = jnp.sum(degt_ref[...], axis=1, keepdims=True) + 1.0
    dinv = lax.rsqrt(deg)
    dinv_ref[...] = dinv
    hw = jnp.dot(h, wc_ref[...], preferred_element_type=jnp.float32) * dinv
    s_ref[0] = hw[:, :_D // 2]
    s_ref[1] = hw[:, _D // 2:]


def _bn_scale_shift(st, gb):
    mu = st[0:1] / float(_N)
    var = st[1:2] / float(_N) - mu * mu
    a = gb[0:1] * lax.rsqrt(var + _EPS)
    return a, gb[1:2] - mu * a


def _phase0(p_ref, dinv_ref, b_ref, y_scr, st_scr):
    i = pl.program_id(1)
    pb = p_ref[...]
    y = jnp.concatenate([pb[0], pb[1]], axis=1) * dinv_ref[...] + b_ref[...]
    y_scr[pl.ds(i * _BR, _BR), :] = y
    rowid = lax.broadcasted_iota(jnp.int32, (_BR, 1), 0) + i * _BR
    ym = jnp.where(rowid < _N, y, 0.0)
    snew = jnp.concatenate([jnp.sum(ym, axis=0, keepdims=True),
                            jnp.sum(ym * ym, axis=0, keepdims=True)], axis=0)

    @pl.when(i == 0)
    def _():
        st_scr[...] = snew

    @pl.when(i > 0)
    def _():
        st_scr[...] = st_scr[...] + snew


def _conv_epi_body(*refs, has_res):
    if has_res:
        (p_ref, dinv_ref, b_ref, gb_ref, wc_ref, hin_ref,
         h_ref, s_ref, y_scr, st_scr) = refs
    else:
        (p_ref, dinv_ref, b_ref, gb_ref, wc_ref,
         h_ref, s_ref, y_scr, st_scr) = refs
    ph = pl.program_id(0)
    i = pl.program_id(1)

    @pl.when(ph == 0)
    def _():
        _phase0(p_ref, dinv_ref, b_ref, y_scr, st_scr)

    @pl.when(ph == 1)
    def _():
        y = y_scr[pl.ds(i * _BR, _BR), :]
        a, c = _bn_scale_shift(st_scr[...], gb_ref[...])
        h = jnp.maximum(y * a + c, 0.0)
        if has_res:
            h = h + hin_ref[...]
        h_ref[...] = h
        hw = jnp.dot(h, wc_ref[...], preferred_element_type=jnp.float32)
        hw = hw * dinv_ref[...]
        s_ref[0] = hw[:, :_D // 2]
        s_ref[1] = hw[:, _D // 2:]


def _conv_head_body(p_ref, dinv_ref, b_ref, gb_ref, hin_ref, w1_ref, b1_ref,
                    w2_ref, b2_ref, o_ref, y_scr, st_scr):
    ph = pl.program_id(0)
    i = pl.program_id(1)

    @pl.when(ph == 0)
    def _():
        _phase0(p_ref, dinv_ref, b_ref, y_scr, st_scr)

    @pl.when(ph == 1)
    def _():
        y = y_scr[pl.ds(i * _BR, _BR), :]
        a, c = _bn_scale_shift(st_scr[...], gb_ref[...])
        h = jnp.maximum(y * a + c, 0.0) + hin_ref[...]
        r = jnp.dot(h, w1_ref[...], preferred_element_type=jnp.float32)
        r = jnp.maximum(r + b1_ref[...], 0.0)
        o = jnp.sum(r * w2_ref[...], axis=1, keepdims=True) + b2_ref[...]
        o_ref[...] = o


def _row_spec(i_map=lambda i: (i, 0), shape=(_BR, _D)):
    return pl.BlockSpec(shape, i_map)


_FULL = lambda i: (0, 0)


def kernel(x, edge_index, W_in, b_in, conv_W, conv_b, bn_g, bn_b,
           Wo1, bo1, Wo2, bo2):
    n, d = x.shape
    e = edge_index.shape[1]
    epw = e // _NW
    assert n == _N and d == _D and e % (_NW * _CHUNK) == 0

    ept = e // _NS
    assert ept % (_CHUNK * 5) == 0
    src = edge_index[0]
    dst = edge_index[1]
    src2 = src.reshape(_NW, epw // _CHUNK, _CHUNK)
    dst2 = dst.reshape(_NW, epw // _CHUNK, _CHUNK)
    src3 = src.reshape(_NS, ept // _CHUNK, _CHUNK)
    dst3 = dst.reshape(_NS, ept // _CHUNK, _CHUNK)
    xp = jnp.pad(x, ((0, _NPAD - n), (0, 0)))
    zerosn = jnp.zeros((_NPAD,), jnp.float32)

    mesh = plsc.VectorSubcoreMesh(core_axis_name="c", subcore_axis_name="s",
                                  num_cores=_NC, num_subcores=_NS)

    deg_call = pl.kernel(
        functools.partial(_deg_body, epw=epw),
        out_type=jax.ShapeDtypeStruct((_NW, _NPAD), jnp.float32),
        mesh=mesh,
        compiler_params=pltpu.CompilerParams(needs_layout_passes=False),
        scratch_types=[
            pltpu.VMEM((epw // _CHUNK, _CHUNK), jnp.int32),
            pltpu.VMEM((_NPAD,), jnp.float32),
        ],
    )
    agg_call = pl.kernel(
        functools.partial(_agg_body, epw=ept),
        out_type=jax.ShapeDtypeStruct((_NC, _NPAD, d // 2), jnp.float32),
        mesh=mesh,
        compiler_params=pltpu.CompilerParams(use_tc_tiling_on_sc=False),
        scratch_types=[
            pltpu.VMEM((ept // _CHUNK, _CHUNK), jnp.int32),
            pltpu.VMEM((ept // _CHUNK, _CHUNK), jnp.int32),
        ] + [pltpu.VMEM((_CHUNK, d // 2), jnp.float32)] * 5 + [
            pltpu.VMEM_SHARED((_NPAD, d // 2), jnp.float32),
            pltpu.SemaphoreType.DMA,
            pltpu.SemaphoreType.DMA,
        ],
    )

    nblk = _NPAD // _BR
    degp = deg_call(dst2, zerosn)
    degt = degp.T

    init_call = pl.pallas_call(
        _init_body,
        grid=(nblk,),
        in_specs=[_row_spec(),
                  pl.BlockSpec((_D, _D), _FULL),
                  pl.BlockSpec((1, _D), _FULL),
                  _row_spec(shape=(_BR, _NW)),
                  pl.BlockSpec((_D, _D), _FULL)],
        out_specs=[_row_spec(),
                   pl.BlockSpec((2, _BR, _D // 2), lambda i: (0, i, 0)),
                   pl.BlockSpec((_BR, 1), lambda i: (i, 0))],
        out_shape=[jax.ShapeDtypeStruct((_NPAD, _D), jnp.float32),
                   jax.ShapeDtypeStruct((2, _NPAD, _D // 2), jnp.float32),
                   jax.ShapeDtypeStruct((_NPAD, 1), jnp.float32)],
    )
    h, s, dinv = init_call(xp, W_in, b_in.reshape(1, _D), degt, conv_W[0])

    dmid = Wo1.shape[1]
    out = None
    for i in range(3):
        p = agg_call(s, src3, dst3)
        gb = jnp.stack([bn_g[i], bn_b[i]])
        base_specs = [
            pl.BlockSpec((_NC, _BR, _D // 2),
                         lambda ph, ib: (0, ib * (1 - ph), 0)),
            pl.BlockSpec((_BR, 1), lambda ph, ib: (ib, 0)),
            pl.BlockSpec((1, _D), lambda ph, ib: (0, 0)),
            pl.BlockSpec((2, _D), lambda ph, ib: (0, 0)),
        ]
        scr = [pltpu.VMEM((_NPAD, _D), jnp.float32),
               pltpu.VMEM((2, _D), jnp.float32)]
        if i < 2:
            has_res = i > 0
            specs = base_specs + [
                pl.BlockSpec((_D, _D), lambda ph, ib: (0, 0))]
            args = [p, dinv, conv_b[i].reshape(1, _D), gb, conv_W[i + 1]]
            if has_res:
                specs.append(pl.BlockSpec((_BR, _D),
                                          lambda ph, ib: (ib * ph, 0)))
                args.append(h)
            fused_call = pl.pallas_call(
                functools.partial(_conv_epi_body, has_res=has_res),
                grid=(2, nblk),
                in_specs=specs,
                out_specs=[pl.BlockSpec((_BR, _D),
                                        lambda ph, ib: (ib * ph, 0)),
                           pl.BlockSpec((2, _BR, _D // 2),
                                        lambda ph, ib: (0, ib * ph, 0))],
                out_shape=[jax.ShapeDtypeStruct((_NPAD, _D), jnp.float32),
                           jax.ShapeDtypeStruct((2, _NPAD, _D // 2),
                                                jnp.float32)],
                scratch_shapes=scr,
            )
            h, s = fused_call(*args)
        else:
            specs = base_specs + [
                pl.BlockSpec((_BR, _D), lambda ph, ib: (ib * ph, 0)),
                pl.BlockSpec((_D, dmid), lambda ph, ib: (0, 0)),
                pl.BlockSpec((1, dmid), lambda ph, ib: (0, 0)),
                pl.BlockSpec((1, dmid), lambda ph, ib: (0, 0)),
                pl.BlockSpec((1, 1), lambda ph, ib: (0, 0))]
            fused_call = pl.pallas_call(
                _conv_head_body,
                grid=(2, nblk),
                in_specs=specs,
                out_specs=[pl.BlockSpec((_BR, 1),
                                        lambda ph, ib: (ib * ph, 0))],
                out_shape=[jax.ShapeDtypeStruct((_NPAD, 1), jnp.float32)],
                scratch_shapes=scr,
            )
            (o2,) = fused_call(p, dinv, conv_b[i].reshape(1, _D), gb, h,
                               Wo1, bo1.reshape(1, dmid),
                               Wo2.reshape(1, dmid), bo2.reshape(1, 1))
            out = o2[:n, 0]
    return out


# speedup vs baseline: 1.0458x; 1.0191x over previous
import functools

import jax
import jax.numpy as jnp
from jax import lax
from jax.experimental import pallas as pl
from jax.experimental.pallas import tpu as pltpu
from jax.experimental.pallas import tpu_sc as plsc

_N = 10000
_NPAD = 10240
_D = 128
_NC = 2
_NS = 16
_NW = _NC * _NS
_CHUNK = 100
_CDEG = 80
_NBUF = 5
_BR = 256
_EPS = 1e-5


def _deg_body(dst2_hbm, zerosn_hbm, out_hbm, didx, hist, *, epw):
    cid = lax.axis_index("c")
    sid = lax.axis_index("s")
    wid = cid * _NS + sid
    nch = epw // _CDEG
    pltpu.sync_copy(dst2_hbm.at[wid], didx)
    pltpu.sync_copy(zerosn_hbm, hist)
    ones16 = jnp.ones((16,), jnp.float32)

    def body(j, carry):
        for k in range(_CDEG // 16):
            idx16 = didx[j, pl.ds(k * 16, 16)]
            plsc.addupdate_scatter(hist, [idx16], ones16)
        return carry

    lax.fori_loop(0, nch, body, 0)
    pltpu.sync_copy(hist, out_hbm.at[wid])


def _agg_body(s2_hbm, src3_hbm, dst3_hbm, out_hbm,
              sidx, didx, r0b, r1b, r2b, r3b, r4b, acc, sem_g, sem_s, *, epw):
    rows = [r0b, r1b, r2b, r3b, r4b]
    nbuf = len(rows)
    lag = 2
    cid = lax.axis_index("c")
    sid = lax.axis_index("s")
    rpt = _NPAD // _NS
    r0 = sid * rpt
    nch = epw // _CHUNK
    half = s2_hbm.at[cid]

    pltpu.sync_copy(half.at[pl.ds(r0, rpt)], acc.at[pl.ds(r0, rpt)])

    pltpu.sync_copy(src3_hbm.at[sid], sidx)
    pltpu.sync_copy(dst3_hbm.at[sid], didx)
    plsc.subcore_barrier()

    for b in range(nbuf - lag):
        pltpu.async_copy(half.at[sidx.at[b]], rows[b], sem_g)

    ngrp = nch // nbuf

    def grp_body(g, carry):
        for b in range(nbuf):
            j = g * nbuf + b
            pltpu.make_async_copy(half.at[sidx.at[b]], rows[b],
                                  sem_g).wait()
            pltpu.async_copy(rows[b], acc.at[didx.at[j]], sem_s, add=True)

            @pl.when(j >= lag)
            def _():
                pltpu.make_async_copy(rows[b], acc.at[didx.at[j]],
                                      sem_s).wait()

            @pl.when(j + nbuf - lag < nch)
            def _():
                nb = (b + nbuf - lag) % nbuf
                pltpu.async_copy(half.at[sidx.at[j + nbuf - lag]], rows[nb],
                                 sem_g)
        return carry

    lax.fori_loop(0, ngrp, grp_body, 0)
    for _ in range(lag):
        pltpu.make_async_copy(rows[0], acc.at[didx.at[0]], sem_s).wait()
    plsc.subcore_barrier()
    pltpu.sync_copy(acc.at[pl.ds(r0, rpt)], out_hbm.at[cid, pl.ds(r0, rpt)])


def _init_body(x_ref, w_ref, b_ref, degt_ref, wc_ref, h_ref, s_ref,
               dinv_ref):
    h = jnp.dot(x_ref[...], w_ref[...], preferred_element_type=jnp.float32)
    h = jnp.maximum(h + b_ref[...], 0.0)
    h_ref[...] = h
    deg = jnp.sum(degt_ref[...], axis=1, keepdims=True) + 1.0
    dinv = lax.rsqrt(deg)
    dinv_ref[...] = dinv
    hw = jnp.dot(h, wc_ref[...], preferred_element_type=jnp.float32) * dinv
    s_ref[0] = hw[:, :_D // 2]
    s_ref[1] = hw[:, _D // 2:]


def _bn_scale_shift(st, gb):
    mu = st[0:1] / float(_N)
    var = st[1:2] / float(_N) - mu * mu
    a = gb[0:1] * lax.rsqrt(var + _EPS)
    return a, gb[1:2] - mu * a


def _phase0(p_ref, dinv_ref, b_ref, y_scr, st_scr):
    i = pl.program_id(1)
    pb = p_ref[...]
    y = jnp.concatenate([pb[0], pb[1]], axis=1) * dinv_ref[...] + b_ref[...]
    y_scr[pl.ds(i * _BR, _BR), :] = y
    rowid = lax.broadcasted_iota(jnp.int32, (_BR, 1), 0) + i * _BR
    ym = jnp.where(rowid < _N, y, 0.0)
    snew = jnp.concatenate([jnp.sum(ym, axis=0, keepdims=True),
                            jnp.sum(ym * ym, axis=0, keepdims=True)], axis=0)

    @pl.when(i == 0)
    def _():
        st_scr[...] = snew

    @pl.when(i > 0)
    def _():
        st_scr[...] = st_scr[...] + snew


def _conv_epi_body(*refs, has_res):
    if has_res:
        (p_ref, dinv_ref, b_ref, gb_ref, wc_ref, hin_ref,
         h_ref, s_ref, y_scr, st_scr) = refs
    else:
        (p_ref, dinv_ref, b_ref, gb_ref, wc_ref,
         h_ref, s_ref, y_scr, st_scr) = refs
    ph = pl.program_id(0)
    i = pl.program_id(1)

    @pl.when(ph == 0)
    def _():
        _phase0(p_ref, dinv_ref, b_ref, y_scr, st_scr)

    @pl.when(ph == 1)
    def _():
        y = y_scr[pl.ds(i * _BR, _BR), :]
        a, c = _bn_scale_shift(st_scr[...], gb_ref[...])
        h = jnp.maximum(y * a + c, 0.0)
        if has_res:
            h = h + hin_ref[...]
        h_ref[...] = h
        hw = jnp.dot(h, wc_ref[...], preferred_element_type=jnp.float32)
        hw = hw * dinv_ref[...]
        s_ref[0] = hw[:, :_D // 2]
        s_ref[1] = hw[:, _D // 2:]


def _conv_head_body(p_ref, dinv_ref, b_ref, gb_ref, hin_ref, w1_ref, b1_ref,
                    w2_ref, b2_ref, o_ref, y_scr, st_scr):
    ph = pl.program_id(0)
    i = pl.program_id(1)

    @pl.when(ph == 0)
    def _():
        _phase0(p_ref, dinv_ref, b_ref, y_scr, st_scr)

    @pl.when(ph == 1)
    def _():
        y = y_scr[pl.ds(i * _BR, _BR), :]
        a, c = _bn_scale_shift(st_scr[...], gb_ref[...])
        h = jnp.maximum(y * a + c, 0.0) + hin_ref[...]
        r = jnp.dot(h, w1_ref[...], preferred_element_type=jnp.float32)
        r = jnp.maximum(r + b1_ref[...], 0.0)
        o = jnp.sum(r * w2_ref[...], axis=1, keepdims=True) + b2_ref[...]
        o_ref[...] = o


def _row_spec(i_map=lambda i: (i, 0), shape=(_BR, _D)):
    return pl.BlockSpec(shape, i_map)


_FULL = lambda i: (0, 0)


def kernel(x, edge_index, W_in, b_in, conv_W, conv_b, bn_g, bn_b,
           Wo1, bo1, Wo2, bo2):
    n, d = x.shape
    e = edge_index.shape[1]
    epw = e // _NW
    assert n == _N and d == _D and e % (_NW * _CDEG) == 0

    ept = e // _NS
    assert ept % (_CHUNK * 5) == 0
    src = edge_index[0]
    dst = edge_index[1]
    dst2 = dst.reshape(_NW, epw // _CDEG, _CDEG)
    src3 = src.reshape(_NS, ept // _CHUNK, _CHUNK)
    dst3 = dst.reshape(_NS, ept // _CHUNK, _CHUNK)
    xp = jnp.pad(x, ((0, _NPAD - n), (0, 0)))
    zerosn = jnp.zeros((_NPAD,), jnp.float32)

    mesh = plsc.VectorSubcoreMesh(core_axis_name="c", subcore_axis_name="s",
                                  num_cores=_NC, num_subcores=_NS)

    deg_call = pl.kernel(
        functools.partial(_deg_body, epw=epw),
        out_type=jax.ShapeDtypeStruct((_NW, _NPAD), jnp.float32),
        mesh=mesh,
        compiler_params=pltpu.CompilerParams(needs_layout_passes=False),
        scratch_types=[
            pltpu.VMEM((epw // _CDEG, _CDEG), jnp.int32),
            pltpu.VMEM((_NPAD,), jnp.float32),
        ],
    )
    agg_call = pl.kernel(
        functools.partial(_agg_body, epw=ept),
        out_type=jax.ShapeDtypeStruct((_NC, _NPAD, d // 2), jnp.float32),
        mesh=mesh,
        compiler_params=pltpu.CompilerParams(use_tc_tiling_on_sc=False),
        scratch_types=[
            pltpu.VMEM((ept // _CHUNK, _CHUNK), jnp.int32),
            pltpu.VMEM((ept // _CHUNK, _CHUNK), jnp.int32),
        ] + [pltpu.VMEM((_CHUNK, d // 2), jnp.float32)] * 5 + [
            pltpu.VMEM_SHARED((_NPAD, d // 2), jnp.float32),
            pltpu.SemaphoreType.DMA,
            pltpu.SemaphoreType.DMA,
        ],
    )

    nblk = _NPAD // _BR
    degp = deg_call(dst2, zerosn)
    degt = degp.T

    init_call = pl.pallas_call(
        _init_body,
        grid=(nblk,),
        in_specs=[_row_spec(),
                  pl.BlockSpec((_D, _D), _FULL),
                  pl.BlockSpec((1, _D), _FULL),
                  _row_spec(shape=(_BR, _NW)),
                  pl.BlockSpec((_D, _D), _FULL)],
        out_specs=[_row_spec(),
                   pl.BlockSpec((2, _BR, _D // 2), lambda i: (0, i, 0)),
                   pl.BlockSpec((_BR, 1), lambda i: (i, 0))],
        out_shape=[jax.ShapeDtypeStruct((_NPAD, _D), jnp.float32),
                   jax.ShapeDtypeStruct((2, _NPAD, _D // 2), jnp.float32),
                   jax.ShapeDtypeStruct((_NPAD, 1), jnp.float32)],
    )
    h, s, dinv = init_call(xp, W_in, b_in.reshape(1, _D), degt, conv_W[0])

    dmid = Wo1.shape[1]
    out = None
    for i in range(3):
        p = agg_call(s, src3, dst3)
        gb = jnp.stack([bn_g[i], bn_b[i]])
        base_specs = [
            pl.BlockSpec((_NC, _BR, _D // 2),
                         lambda ph, ib: (0, ib * (1 - ph), 0)),
            pl.BlockSpec((_BR, 1), lambda ph, ib: (ib, 0)),
            pl.BlockSpec((1, _D), lambda ph, ib: (0, 0)),
            pl.BlockSpec((2, _D), lambda ph, ib: (0, 0)),
        ]
        scr = [pltpu.VMEM((_NPAD, _D), jnp.float32),
               pltpu.VMEM((2, _D), jnp.float32)]
        if i < 2:
            has_res = i > 0
            specs = base_specs + [
                pl.BlockSpec((_D, _D), lambda ph, ib: (0, 0))]
            args = [p, dinv, conv_b[i].reshape(1, _D), gb, conv_W[i + 1]]
            if has_res:
                specs.append(pl.BlockSpec((_BR, _D),
                                          lambda ph, ib: (ib * ph, 0)))
                args.append(h)
            fused_call = pl.pallas_call(
                functools.partial(_conv_epi_body, has_res=has_res),
                grid=(2, nblk),
                in_specs=specs,
                out_specs=[pl.BlockSpec((_BR, _D),
                                        lambda ph, ib: (ib * ph, 0)),
                           pl.BlockSpec((2, _BR, _D // 2),
                                        lambda ph, ib: (0, ib * ph, 0))],
                out_shape=[jax.ShapeDtypeStruct((_NPAD, _D), jnp.float32),
                           jax.ShapeDtypeStruct((2, _NPAD, _D // 2),
                                                jnp.float32)],
                scratch_shapes=scr,
            )
            h, s = fused_call(*args)
        else:
            specs = base_specs + [
                pl.BlockSpec((_BR, _D), lambda ph, ib: (ib * ph, 0)),
                pl.BlockSpec((_D, dmid), lambda ph, ib: (0, 0)),
                pl.BlockSpec((1, dmid), lambda ph, ib: (0, 0)),
                pl.BlockSpec((1, dmid), lambda ph, ib: (0, 0)),
                pl.BlockSpec((1, 1), lambda ph, ib: (0, 0))]
            fused_call = pl.pallas_call(
                _conv_head_body,
                grid=(2, nblk),
                in_specs=specs,
                out_specs=[pl.BlockSpec((_BR, 1),
                                        lambda ph, ib: (ib * ph, 0))],
                out_shape=[jax.ShapeDtypeStruct((_NPAD, 1), jnp.float32)],
                scratch_shapes=scr,
            )
            (o2,) = fused_call(p, dinv, conv_b[i].reshape(1, _D), gb, h,
                               Wo1, bo1.reshape(1, dmid),
                               Wo2.reshape(1, dmid), bo2.reshape(1, 1))
            out = o2[:n, 0]
    return out
